# baseline (speedup 1.0000x reference)
import functools

import jax
import jax.numpy as jnp
from jax import lax
from jax.experimental import pallas as pl
from jax.experimental.pallas import tpu as pltpu
from jax.experimental.pallas import tpu_sc as plsc

_EMBED = 32
_PADW = 128
_NC = 2
_NS = 16
_NW = _NC * _NS
_CHUNK = 128


def _make_gather2(B):
    bpw = B // _NW
    nchunk = bpw // _CHUNK
    mesh = plsc.VectorSubcoreMesh(core_axis_name="c", subcore_axis_name="s")

    @functools.partial(
        pl.kernel,
        mesh=mesh,
        compiler_params=pltpu.CompilerParams(use_tc_tiling_on_sc=False),
        out_type=jax.ShapeDtypeStruct((B, 2 * _EMBED), jnp.float32),
        scratch_types=[
            pltpu.VMEM((nchunk, _CHUNK), jnp.int32),
            pltpu.VMEM((2, _CHUNK, _PADW), jnp.float32),
            pltpu.VMEM((2, _CHUNK, _PADW), jnp.float32),
            pltpu.SemaphoreType.DMA,
            pltpu.SemaphoreType.DMA,
            pltpu.SemaphoreType.DMA,
        ],
    )
    def gather2(idx_hbm, ct_hbm, rt_hbm, out_hbm,
                idx_v, rows_c, rows_r, sem_c, sem_r, sem_w):
        wid = lax.axis_index("s") * _NC + lax.axis_index("c")
        base = wid * bpw
        pltpu.sync_copy(idx_hbm.at[wid], idx_v)
        gc = [None] * nchunk
        gr = [None] * nchunk
        wc = [None] * nchunk
        wr = [None] * nchunk

        def fire_writes(p):
            s = p % 2
            gc[p].wait()
            wc[p] = pltpu.async_copy(
                rows_c.at[s, :, pl.ds(0, _EMBED)],
                out_hbm.at[pl.ds(base + p * _CHUNK, _CHUNK), pl.ds(0, _EMBED)],
                sem_w)
            gr[p].wait()
            wr[p] = pltpu.async_copy(
                rows_r.at[s, :, pl.ds(0, _EMBED)],
                out_hbm.at[pl.ds(base + p * _CHUNK, _CHUNK),
                           pl.ds(_EMBED, _EMBED)],
                sem_w)

        for j in range(nchunk):
            s = j % 2
            if j >= 2:
                wc[j - 2].wait()
                wr[j - 2].wait()
            gc[j] = pltpu.async_copy(ct_hbm.at[idx_v.at[j]], rows_c.at[s], sem_c)
            gr[j] = pltpu.async_copy(rt_hbm.at[idx_v.at[j]], rows_r.at[s], sem_r)
            if j >= 1:
                fire_writes(j - 1)
        fire_writes(nchunk - 1)
        for p in (nchunk - 2, nchunk - 1):
            wc[p].wait()
            wr[p].wait()

    return gather2


_B = 1024 * 20
_GATHER2 = _make_gather2(_B)


def _tc_copy_kernel(in_ref, out_ref):
    out_ref[...] = in_ref[...]


def _tc_copy_t(table_t):
    d, v = table_t.shape
    blk = 8
    return pl.pallas_call(
        _tc_copy_kernel,
        grid=(d // blk,),
        in_specs=[pl.BlockSpec((blk, v), lambda i: (i, 0))],
        out_specs=pl.BlockSpec((blk, v), lambda i: (i, 0)),
        out_shape=jax.ShapeDtypeStruct((d, v), table_t.dtype),
    )(table_t)


def kernel(program, concept_table, relation_table):
    batch, prog_len = program.shape[0], program.shape[1]
    idx = program[:, :, 1].astype(jnp.int32).reshape(_NW, -1, _CHUNK)
    pad = ((0, 0), (0, _PADW - _EMBED))
    ct_p = jnp.pad(concept_table, pad)
    rt_p = jnp.pad(relation_table, pad)
    out = _GATHER2(idx, ct_p, rt_p)
    out = out.reshape(batch, prog_len, 2 * _EMBED)
    all_concepts = _tc_copy_t(concept_table.T).T
    return out, all_concepts
